# baseline (speedup 1.0000x reference)
import functools

import jax
import jax.numpy as jnp
from jax.experimental import pallas as pl

_Q = 1024
_D = 512
_BQ = 1024
_DEF = jax.lax.Precision.DEFAULT


def _retrieve(qb, p):
    sim = jax.lax.dot_general(
        qb, p, (((1,), (1,)), ((), ())), preferred_element_type=jnp.float32,
        precision=_DEF)
    m = jnp.max(sim, axis=-1, keepdims=True)
    e = jnp.exp((sim - m).astype(jnp.bfloat16))
    s = jnp.sum(e, axis=-1, keepdims=True, dtype=jnp.float32)
    num = jax.lax.dot_general(
        e, p, (((1,), (0,)), ((), ())),
        preferred_element_type=jnp.float32, precision=_DEF)
    return num * (1.0 / s)


def _partial(qb, p):
    sim = jax.lax.dot_general(
        qb, p, (((1,), (1,)), ((), ())), preferred_element_type=jnp.float32,
        precision=_DEF)
    m = jnp.max(sim, axis=-1, keepdims=True)
    e = jnp.exp((sim - m).astype(jnp.bfloat16))
    s = jnp.sum(e, axis=-1, keepdims=True, dtype=jnp.float32)
    num = jax.lax.dot_general(
        e, p, (((1,), (0,)), ((), ())),
        preferred_element_type=jnp.float32, precision=_DEF)
    return m, s, num


_CHUNKS = ((0, 1280), (1280, 2560), (2560, 3840), (3840, 5000))


def _retrieve_global(qb, pg_ref):
    parts = [_partial(qb, pg_ref[a:b, :].astype(jnp.bfloat16))
             for a, b in _CHUNKS]
    mg = parts[0][0]
    for m, _, _ in parts[1:]:
        mg = jnp.maximum(mg, m)
    s = jnp.zeros((), jnp.float32)
    num = jnp.zeros((), jnp.float32)
    for m, sc, nc in parts:
        w = jnp.exp(m - mg)
        s = s + sc * w
        num = num + nc * w
    return num * (1.0 / s)


def _body(qb_ref, pg_ref, pa_ref, pb_ref, w1_ref, b1_ref, w2t_ref, b2_ref,
          o_ref):
    qb = qb_ref[...].astype(jnp.bfloat16)
    rg = _retrieve_global(qb, pg_ref)
    ra = _retrieve(qb, pa_ref[...].astype(jnp.bfloat16))
    rb = _retrieve(qb, pb_ref[...].astype(jnp.bfloat16))
    cr = 0.5 * (ra + rb)

    comb = jnp.concatenate([cr, rg], axis=-1)
    h = jax.lax.dot_general(
        comb.astype(jnp.bfloat16), w1_ref[...].astype(jnp.bfloat16),
        (((1,), (0,)), ((), ())),
        preferred_element_type=jnp.float32, precision=_DEF) + b1_ref[...]
    h = 0.5 * h * (1.0 + jax.lax.erf(h * 0.7071067811865476))
    hw = (h * w2t_ref[...]).astype(jnp.bfloat16)
    logit = jax.lax.dot_general(
        hw, jnp.ones((64, _D), jnp.bfloat16), (((1,), (0,)), ((), ())),
        preferred_element_type=jnp.float32, precision=_DEF) + b2_ref[...]
    gate = jax.nn.sigmoid(logit)
    o_ref[...] = gate * cr + (1.0 - gate) * rg


@functools.partial(jax.jit, static_argnames=())
def kernel(query, global_patterns, classA_patterns, classB_patterns,
           W1, b1, W2, b2):
    kg = global_patterns.shape[0]
    kc = classA_patterns.shape[0]
    grid = (_Q // _BQ,)
    out = pl.pallas_call(
        _body,
        grid=grid,
        in_specs=[
            pl.BlockSpec((_BQ, _D), lambda i: (i, 0)),
            pl.BlockSpec((kg, _D), lambda i: (0, 0)),
            pl.BlockSpec((kc, _D), lambda i: (0, 0)),
            pl.BlockSpec((kc, _D), lambda i: (0, 0)),
            pl.BlockSpec((2 * _D, 64), lambda i: (0, 0)),
            pl.BlockSpec((1, 64), lambda i: (0, 0)),
            pl.BlockSpec((1, 64), lambda i: (0, 0)),
            pl.BlockSpec((1, 1), lambda i: (0, 0)),
        ],
        out_specs=pl.BlockSpec((_BQ, _D), lambda i: (i, 0)),
        out_shape=jax.ShapeDtypeStruct((_Q, _D), jnp.float32),
    )(query, global_patterns, classA_patterns, classB_patterns,
      W1, b1.reshape(1, 64), W2.reshape(1, 64), b2.reshape(1, 1))
    return out


# speedup vs baseline: 1.0542x; 1.0542x over previous
import functools

import jax
import jax.numpy as jnp
from jax.experimental import pallas as pl

_Q = 1024
_D = 512
_BQ = 1024
_DEF = jax.lax.Precision.DEFAULT


def _retrieve(qb, p):
    sim = jax.lax.dot_general(
        qb, p, (((1,), (1,)), ((), ())), preferred_element_type=jnp.float32,
        precision=_DEF)
    m = jnp.max(sim, axis=-1, keepdims=True)
    e = jnp.exp((sim - m).astype(jnp.bfloat16))
    s = jnp.sum(e, axis=-1, keepdims=True, dtype=jnp.float32)
    num = jax.lax.dot_general(
        e, p, (((1,), (0,)), ((), ())),
        preferred_element_type=jnp.float32, precision=_DEF)
    return num * (1.0 / s)


def _body(qb_ref, pg_ref, pa_ref, pb_ref, w1_ref, b1_ref, w2t_ref, b2_ref,
          o_ref):
    qb = qb_ref[...].astype(jnp.bfloat16)
    rg = _retrieve(qb, pg_ref[...].astype(jnp.bfloat16))
    ra = _retrieve(qb, pa_ref[...].astype(jnp.bfloat16))
    rb = _retrieve(qb, pb_ref[...].astype(jnp.bfloat16))
    cr = 0.5 * (ra + rb)

    comb = jnp.concatenate([cr, rg], axis=-1)
    h = jax.lax.dot_general(
        comb.astype(jnp.bfloat16), w1_ref[...].astype(jnp.bfloat16),
        (((1,), (0,)), ((), ())),
        preferred_element_type=jnp.float32, precision=_DEF) + b1_ref[...]
    h = 0.5 * h * (1.0 + jax.lax.erf(h * 0.7071067811865476))
    hw = (h * w2t_ref[...]).astype(jnp.bfloat16)
    logit = jax.lax.dot_general(
        hw, jnp.ones((64, _D), jnp.bfloat16), (((1,), (0,)), ((), ())),
        preferred_element_type=jnp.float32, precision=_DEF) + b2_ref[...]
    gate = jax.nn.sigmoid(logit)
    o_ref[...] = gate * cr + (1.0 - gate) * rg


@functools.partial(jax.jit, static_argnames=())
def kernel(query, global_patterns, classA_patterns, classB_patterns,
           W1, b1, W2, b2):
    kg = global_patterns.shape[0]
    kc = classA_patterns.shape[0]
    grid = (_Q // _BQ,)
    out = pl.pallas_call(
        _body,
        grid=grid,
        in_specs=[
            pl.BlockSpec((_BQ, _D), lambda i: (i, 0)),
            pl.BlockSpec((kg, _D), lambda i: (0, 0)),
            pl.BlockSpec((kc, _D), lambda i: (0, 0)),
            pl.BlockSpec((kc, _D), lambda i: (0, 0)),
            pl.BlockSpec((2 * _D, 64), lambda i: (0, 0)),
            pl.BlockSpec((1, 64), lambda i: (0, 0)),
            pl.BlockSpec((1, 64), lambda i: (0, 0)),
            pl.BlockSpec((1, 1), lambda i: (0, 0)),
        ],
        out_specs=pl.BlockSpec((_BQ, _D), lambda i: (i, 0)),
        out_shape=jax.ShapeDtypeStruct((_Q, _D), jnp.float32),
    )(query, global_patterns, classA_patterns, classB_patterns,
      W1, b1.reshape(1, 64), W2.reshape(1, 64), b2.reshape(1, 1))
    return out
